# baseline (speedup 1.0000x reference)
import functools

import jax
import jax.numpy as jnp
from jax import lax
from jax.experimental import pallas as pl
from jax.experimental.pallas import tpu as pltpu
from jax.experimental.pallas import tpu_sc as plsc

N = 10000
E = 320000
D = 128
DH = D // 2
EPS = 1e-5

NC = 2
NS = 16

CHA = 125
EPT = E // NS
NCHT = EPT // CHA
DEG_G = 8
ZT = 10
RPZ = N // ZT
SRW = 200
NBUF = 4
PF = 2
NQ = 4
QCH = NCHT // NQ
XB = 128

_sc_mesh = plsc.VectorSubcoreMesh(core_axis_name="c", subcore_axis_name="s")


def _rsqrt16(x16):
    s = x16
    for _ in range(15):
        s = 0.5 * (s + x16 / s)
    return 1.0 / s


@functools.partial(
    pl.kernel,
    out_type=(
        jax.ShapeDtypeStruct((NC * N, DH), jnp.float32),
        jax.ShapeDtypeStruct((N, DH), jnp.float32),
        jax.ShapeDtypeStruct((N, DH), jnp.float32),
        jax.ShapeDtypeStruct((NC * N,), jnp.float32),
    ),
    mesh=_sc_mesh,
    scratch_types=[
        pltpu.VMEM((QCH, CHA), jnp.int32),
        pltpu.VMEM((QCH, CHA), jnp.int32),
        pltpu.VMEM((NBUF, CHA, DH), jnp.float32),
        pltpu.VMEM((SRW, DH), jnp.float32),
        pltpu.VMEM((128,), jnp.float32),
        pltpu.VMEM((1000,), jnp.float32),
        pltpu.VMEM((RPZ + 8, ), jnp.float32),
        pltpu.VMEM((XB, DH), jnp.float32),
        pltpu.VMEM_SHARED((N, DH), jnp.float32),
        pltpu.VMEM_SHARED((N,), jnp.float32),
        pltpu.VMEM_SHARED((N,), jnp.float32),
    ] + [pltpu.SemaphoreType.DMA] * (2 * NBUF + 1),
    compiler_params=pltpu.CompilerParams(use_tc_tiling_on_sc=False),
)
def _gcn_sc_kernel(x, src2, dst2, zh, z1, part, flo, fhi, dsth,
                   sidx_v, didx_v, rows_v, stage_v, ones_v, zv, nrm_v,
                   xblk_v, agg_s, deg_s, dsth_s, *sems):
    gsem = sems[:NBUF]
    ssem = sems[NBUF:2 * NBUF]
    hsem = sems[2 * NBUF]
    c = lax.axis_index("c")
    s = lax.axis_index("s")

    for i in range(128 // 16):
        ones_v[pl.ds(i * 16, 16)] = jnp.ones((16,), jnp.float32)
    ones_r = ones_v.at[pl.ds(0, CHA)]

    @pl.when(s < ZT)
    def _():
        pltpu.sync_copy(z1.at[pl.ds(s * 1000, 1000)], zv)
        pltpu.sync_copy(zv, deg_s.at[pl.ds(s * 1000, 1000)])
        pltpu.sync_copy(zv, dsth_s.at[pl.ds(s * 1000, 1000)])
        for r in range(RPZ // SRW):
            off = pl.ds(s * RPZ + r * SRW, SRW)
            pltpu.sync_copy(zh.at[off], stage_v)
            pltpu.sync_copy(stage_v, agg_s.at[off])
    plsc.subcore_barrier()

    def hist_pass(idx2, row0, nrows, table):
        for q in range(nrows // QCH):
            pltpu.sync_copy(idx2.at[pl.ds(row0 + q * QCH, QCH)], sidx_v)

            @pl.loop(0, QCH, step=DEG_G)
            def _(jb):
                for g in range(DEG_G):
                    pltpu.async_copy(ones_r, table.at[sidx_v.at[jb + g]],
                                     hsem, add=True)
                for g in range(DEG_G):
                    pltpu.make_async_copy(ones_r, table.at[sidx_v.at[jb + g]],
                                          hsem).wait()

    hist_pass(src2, s * NCHT, NCHT, deg_s)
    hist_pass(dst2, s * NCHT + c * (NCHT // 2), NCHT // 2, dsth_s)
    plsc.subcore_barrier()

    def scale_half(fout):
        col0 = c * DH
        pltpu.sync_copy(deg_s.at[pl.ds(s * RPZ, RPZ)],
                        nrm_v.at[pl.ds(0, RPZ)])

        @pl.loop(0, (RPZ + 8) // 16)
        def _(k):
            dv = jnp.maximum(nrm_v[pl.ds(k * 16, 16)], 1.0)
            nrm_v[pl.ds(k * 16, 16)] = _rsqrt16(dv)

        for blk in range(8):
            nrows = XB if blk < 7 else RPZ - 7 * XB
            row0 = s * RPZ + blk * XB
            pltpu.sync_copy(x.at[pl.ds(row0, nrows), pl.ds(col0, DH)],
                            xblk_v.at[pl.ds(0, nrows)])

            def rowgroup(g, cnt):
                nv16 = nrm_v[pl.ds(blk * XB + g * 16, 16)]
                for i in range(cnt):
                    vecn = lax.broadcast(nv16[i], (16,))
                    for q in range(DH // 16):
                        sl = pl.ds(q * 16, 16)
                        xblk_v[g * 16 + i, sl] = xblk_v[g * 16 + i, sl] * vecn

            @pl.loop(0, nrows // 16)
            def _(g):
                rowgroup(g, 16)
            if nrows % 16:
                rowgroup(nrows // 16, nrows % 16)
            pltpu.sync_copy(xblk_v.at[pl.ds(0, nrows)],
                            fout.at[pl.ds(row0, nrows)])

    @pl.when(s < ZT)
    def _():
        @pl.when(c == 0)
        def _():
            scale_half(flo)

        @pl.when(c == 1)
        def _():
            scale_half(fhi)
    plsc.subcore_barrier()

    def edge_pass(ftab):
        def start_gather(j, b):
            pltpu.async_copy(ftab.at[sidx_v.at[j]], rows_v.at[b], gsem[b])

        def wait_gather(j, b):
            pltpu.make_async_copy(ftab.at[sidx_v.at[j]], rows_v.at[b],
                                  gsem[b]).wait()

        def start_scatter(j, b):
            pltpu.async_copy(rows_v.at[b], agg_s.at[didx_v.at[j]], ssem[b],
                             add=True)

        def wait_scatter(j, b):
            pltpu.make_async_copy(rows_v.at[b], agg_s.at[didx_v.at[j]],
                                  ssem[b]).wait()

        for q in range(NQ):
            qoff = pl.ds(s * NCHT + q * QCH, QCH)
            pltpu.sync_copy(src2.at[qoff], sidx_v)
            pltpu.sync_copy(dst2.at[qoff], didx_v)

            for b in range(PF):
                start_gather(b, b)

            @pl.loop(0, QCH, step=NBUF)
            def _(jb):
                for bb in range(NBUF):
                    j = jb + bb
                    nb = (bb + PF) % NBUF
                    @pl.when(j + PF < QCH)
                    def _(j=j, nb=nb):
                        @pl.when(j + PF - NBUF >= 0)
                        def _():
                            wait_scatter(j + PF - NBUF, nb)
                        start_gather(j + PF, nb)
                    wait_gather(j, bb)
                    start_scatter(j, bb)

            for bb in range(NBUF):
                j = QCH - NBUF + bb
                wait_scatter(j, j % NBUF)

    @pl.when(c == 0)
    def _():
        edge_pass(flo)

    @pl.when(c == 1)
    def _():
        edge_pass(fhi)
    plsc.subcore_barrier()

    @pl.when(s < ZT)
    def _():
        for r in range(RPZ // SRW):
            pltpu.sync_copy(agg_s.at[pl.ds(s * RPZ + r * SRW, SRW)], stage_v)
            pltpu.sync_copy(stage_v,
                            part.at[pl.ds(c * N + s * RPZ + r * SRW, SRW)])
        pltpu.sync_copy(dsth_s.at[pl.ds(s * 1000, 1000)], zv)
        pltpu.sync_copy(zv, dsth.at[pl.ds(c * N + s * 1000, 1000)])


def _final_body(p_ref, d_ref, w_ref, b_ref, x_ref, g_ref, be_ref, o_ref):
    rst = (jnp.dot(p_ref[0], w_ref[0], preferred_element_type=jnp.float32)
           + jnp.dot(p_ref[1], w_ref[1], preferred_element_type=jnp.float32))
    ndst = lax.rsqrt(jnp.maximum(d_ref[0] + d_ref[1], 1.0))
    rst = rst * ndst + b_ref[...]
    mean = jnp.mean(rst, axis=0, keepdims=True)
    var = jnp.mean(rst * rst, axis=0, keepdims=True) - mean * mean
    o_ref[...] = ((rst - mean) * lax.rsqrt(var + EPS) * g_ref[...]
                  + be_ref[...] + x_ref[...])


_final_call = pl.pallas_call(
    _final_body,
    out_shape=jax.ShapeDtypeStruct((N, D), jnp.float32),
)


def kernel(x, edge_index, W, b, gamma, beta):
    src2 = edge_index[0].reshape(E // CHA, CHA)
    dst2 = edge_index[1].reshape(E // CHA, CHA)
    z1 = jnp.zeros((N,), jnp.float32)
    zh = jnp.zeros((N, DH), jnp.float32)

    part, _, _, dsth = _gcn_sc_kernel(x, src2, dst2, zh, z1)
    w2 = W.reshape(2, DH, D)
    return _final_call(part.reshape(NC, N, DH), dsth.reshape(NC, N, 1),
                       w2, b.reshape(1, D), x,
                       gamma.reshape(1, D), beta.reshape(1, D))


# speedup vs baseline: 1.1496x; 1.1496x over previous
import functools

import jax
import jax.numpy as jnp
from jax import lax
from jax.experimental import pallas as pl
from jax.experimental.pallas import tpu as pltpu
from jax.experimental.pallas import tpu_sc as plsc

N = 10000
E = 320000
D = 128
DH = D // 2
EPS = 1e-5

NC = 2
NS = 16

_sc_mesh = plsc.VectorSubcoreMesh(core_axis_name="c", subcore_axis_name="s")

CHA = 125
EPT = E // NS
NCHT = EPT // CHA
DEG_G = 8


ZT = 10
RPZ = N // ZT
XB = 128


def _rsqrt16(x16):
    s = x16
    for _ in range(15):
        s = 0.5 * (s + x16 / s)
    return 1.0 / s


@functools.partial(
    pl.kernel,
    out_type=(
        jax.ShapeDtypeStruct((NC * N,), jnp.float32),
        jax.ShapeDtypeStruct((N, DH), jnp.float32),
        jax.ShapeDtypeStruct((N, DH), jnp.float32),
    ),
    mesh=_sc_mesh,
    scratch_types=[
        pltpu.VMEM((NCHT, CHA), jnp.int32),
        pltpu.VMEM((128,), jnp.float32),
        pltpu.VMEM((1000,), jnp.float32),
        pltpu.VMEM((N,), jnp.float32),
        pltpu.VMEM((RPZ + 8,), jnp.float32),
        pltpu.VMEM((XB, D), jnp.float32),
        pltpu.VMEM_SHARED((N,), jnp.float32),
        pltpu.SemaphoreType.DMA,
    ],
    compiler_params=pltpu.CompilerParams(use_tc_tiling_on_sc=False),
)
def _deg_kernel(x, src2, dst2, z1, deg_out, flo, fhi,
                idx_v, ones_v, zv, dv, nrm_v, xblk_v, deg_s, sem):
    c = lax.axis_index("c")
    s = lax.axis_index("s")
    for i in range(128 // 16):
        ones_v[pl.ds(i * 16, 16)] = jnp.ones((16,), jnp.float32)
    ones_r = ones_v.at[pl.ds(0, CHA)]
    @pl.when(c == 0)
    def _():
        pltpu.sync_copy(src2.at[pl.ds(s * NCHT, NCHT)], idx_v)

    @pl.when(c == 1)
    def _():
        pltpu.sync_copy(dst2.at[pl.ds(s * NCHT, NCHT)], idx_v)
    @pl.when(s < 10)
    def _():
        pltpu.sync_copy(z1.at[pl.ds(s * 1000, 1000)], zv)
        pltpu.sync_copy(zv, deg_s.at[pl.ds(s * 1000, 1000)])
    plsc.subcore_barrier()

    @pl.loop(0, NCHT, step=DEG_G)
    def _(jb):
        for g in range(DEG_G):
            pltpu.async_copy(ones_r, deg_s.at[idx_v.at[jb + g]], sem, add=True)
        for g in range(DEG_G):
            pltpu.make_async_copy(ones_r, deg_s.at[idx_v.at[jb + g]], sem).wait()

    plsc.subcore_barrier()

    @pl.when(s == 0)
    def _():
        pltpu.sync_copy(deg_s, dv)
        pltpu.sync_copy(dv, deg_out.at[pl.ds(c * N, N)])

    @pl.when(jnp.logical_and(c == 0, s < ZT))
    def _():
        pltpu.sync_copy(deg_s.at[pl.ds(s * RPZ, RPZ)],
                        nrm_v.at[pl.ds(0, RPZ)])

        @pl.loop(0, (RPZ + 8) // 16)
        def _(k):
            dvv = jnp.maximum(nrm_v[pl.ds(k * 16, 16)], 1.0)
            nrm_v[pl.ds(k * 16, 16)] = _rsqrt16(dvv)

        for blk in range(8):
            nrows = XB if blk < 7 else RPZ - 7 * XB
            row0 = s * RPZ + blk * XB
            pltpu.sync_copy(x.at[pl.ds(row0, nrows)],
                            xblk_v.at[pl.ds(0, nrows)])

            def rowgroup(g, cnt):
                nv16 = nrm_v[pl.ds(blk * XB + g * 16, 16)]
                for i in range(cnt):
                    vecn = lax.broadcast(nv16[i], (16,))
                    for q in range(D // 16):
                        sl = pl.ds(q * 16, 16)
                        xblk_v[g * 16 + i, sl] = xblk_v[g * 16 + i, sl] * vecn

            @pl.loop(0, nrows // 16)
            def _(g):
                rowgroup(g, 16)
            if nrows % 16:
                rowgroup(nrows // 16, nrows % 16)
            pltpu.sync_copy(xblk_v.at[pl.ds(0, nrows), pl.ds(0, DH)],
                            flo.at[pl.ds(row0, nrows)])
            pltpu.sync_copy(xblk_v.at[pl.ds(0, nrows), pl.ds(DH, DH)],
                            fhi.at[pl.ds(row0, nrows)])


SRW = 200
NBUF = 4
PF = 2
NQ = 1
QCH = NCHT // NQ


@functools.partial(
    pl.kernel,
    out_type=jax.ShapeDtypeStruct((NC * N, DH), jnp.float32),
    mesh=_sc_mesh,
    scratch_types=[
        pltpu.VMEM((QCH, CHA), jnp.int32),
        pltpu.VMEM((QCH, CHA), jnp.int32),
        pltpu.VMEM((NBUF, CHA, DH), jnp.float32),
        pltpu.VMEM((SRW, DH), jnp.float32),
        pltpu.VMEM_SHARED((N, DH), jnp.float32),
    ] + [pltpu.SemaphoreType.DMA] * (2 * NBUF),
    compiler_params=pltpu.CompilerParams(use_tc_tiling_on_sc=False),
)
def _agg_kernel(feat_lo, feat_hi, src2, dst2, zh, part,
                sidx_v, didx_v, rows_v, stage_v, agg_s, *sems):
    gsem = sems[:NBUF]
    ssem = sems[NBUF:]
    c = lax.axis_index("c")
    s = lax.axis_index("s")
    @pl.when(s < ZT)
    def _():
        for r in range(RPZ // SRW):
            off = pl.ds(s * RPZ + r * SRW, SRW)
            pltpu.sync_copy(zh.at[off], stage_v)
            pltpu.sync_copy(stage_v, agg_s.at[off])
    plsc.subcore_barrier()

    def edge_pass(ftab):
        def start_gather(j, b):
            pltpu.async_copy(ftab.at[sidx_v.at[j]], rows_v.at[b], gsem[b])

        def wait_gather(j, b):
            pltpu.make_async_copy(ftab.at[sidx_v.at[j]], rows_v.at[b],
                                  gsem[b]).wait()

        def start_scatter(j, b):
            pltpu.async_copy(rows_v.at[b], agg_s.at[didx_v.at[j]], ssem[b],
                             add=True)

        def wait_scatter(j, b):
            pltpu.make_async_copy(rows_v.at[b], agg_s.at[didx_v.at[j]],
                                  ssem[b]).wait()

        for q in range(NQ):
            qoff = pl.ds(s * NCHT + q * QCH, QCH)
            pltpu.sync_copy(src2.at[qoff], sidx_v)
            pltpu.sync_copy(dst2.at[qoff], didx_v)

            for b in range(PF):
                start_gather(b, b)

            @pl.loop(0, QCH, step=NBUF)
            def _(jb):
                for bb in range(NBUF):
                    j = jb + bb
                    nb = (bb + PF) % NBUF
                    @pl.when(j + PF < QCH)
                    def _(j=j, nb=nb):
                        @pl.when(j + PF - NBUF >= 0)
                        def _():
                            wait_scatter(j + PF - NBUF, nb)
                        start_gather(j + PF, nb)
                    wait_gather(j, bb)
                    start_scatter(j, bb)

            for bb in range(NBUF):
                j = QCH - NBUF + bb
                wait_scatter(j, j % NBUF)

    @pl.when(c == 0)
    def _():
        edge_pass(feat_lo)

    @pl.when(c == 1)
    def _():
        edge_pass(feat_hi)

    plsc.subcore_barrier()

    @pl.when(s < ZT)
    def _():
        for r in range(RPZ // SRW):
            pltpu.sync_copy(agg_s.at[pl.ds(s * RPZ + r * SRW, SRW)], stage_v)
            pltpu.sync_copy(stage_v,
                            part.at[pl.ds(c * N + s * RPZ + r * SRW, SRW)])


def _final_body(p_ref, w_ref, b_ref, x_ref, ddst_ref, g_ref, be_ref, o_ref):
    rst = (jnp.dot(p_ref[0], w_ref[0], preferred_element_type=jnp.float32)
           + jnp.dot(p_ref[1], w_ref[1], preferred_element_type=jnp.float32))
    ndst = lax.rsqrt(jnp.maximum(ddst_ref[...], 1.0))
    rst = rst * ndst + b_ref[...]
    mean = jnp.mean(rst, axis=0, keepdims=True)
    var = jnp.mean(rst * rst, axis=0, keepdims=True) - mean * mean
    o_ref[...] = ((rst - mean) * lax.rsqrt(var + EPS) * g_ref[...]
                  + be_ref[...] + x_ref[...])


_final_call = pl.pallas_call(
    _final_body,
    out_shape=jax.ShapeDtypeStruct((N, D), jnp.float32),
)


def kernel(x, edge_index, W, b, gamma, beta):
    src2 = edge_index[0].reshape(E // CHA, CHA)
    dst2 = edge_index[1].reshape(E // CHA, CHA)
    z1 = jnp.zeros((N,), jnp.float32)
    zh = jnp.zeros((N, DH), jnp.float32)

    deg, flo, fhi = _deg_kernel(x, src2, dst2, z1)
    ddst = deg[N:].reshape(N, 1)
    part = _agg_kernel(flo, fhi, src2, dst2, zh).reshape(NC, N, DH)
    w2 = W.reshape(2, DH, D)
    return _final_call(part, w2, b.reshape(1, D), x, ddst,
                       gamma.reshape(1, D), beta.reshape(1, D))


# speedup vs baseline: 1.1768x; 1.0237x over previous
import functools

import jax
import jax.numpy as jnp
from jax import lax
from jax.experimental import pallas as pl
from jax.experimental.pallas import tpu as pltpu
from jax.experimental.pallas import tpu_sc as plsc

N = 10000
E = 320000
D = 128
DH = D // 2
EPS = 1e-5

NC = 2
NS = 16

_sc_mesh = plsc.VectorSubcoreMesh(core_axis_name="c", subcore_axis_name="s")

CHA = 125
EPT = E // NS
NCHT = EPT // CHA
DEG_G = 8


@functools.partial(
    pl.kernel,
    out_type=jax.ShapeDtypeStruct((NC * N,), jnp.float32),
    mesh=_sc_mesh,
    scratch_types=[
        pltpu.VMEM((NCHT, CHA), jnp.int32),
        pltpu.VMEM((128,), jnp.float32),
        pltpu.VMEM((1000,), jnp.float32),
        pltpu.VMEM((N,), jnp.float32),
        pltpu.VMEM_SHARED((N,), jnp.float32),
        pltpu.SemaphoreType.DMA,
    ],
    compiler_params=pltpu.CompilerParams(use_tc_tiling_on_sc=False),
)
def _deg_kernel(src2, dst2, z1, deg_out, idx_v, ones_v, zv, dv, deg_s, sem):
    c = lax.axis_index("c")
    s = lax.axis_index("s")
    for i in range(128 // 16):
        ones_v[pl.ds(i * 16, 16)] = jnp.ones((16,), jnp.float32)
    ones_r = ones_v.at[pl.ds(0, CHA)]
    @pl.when(c == 0)
    def _():
        pltpu.sync_copy(src2.at[pl.ds(s * NCHT, NCHT)], idx_v)

    @pl.when(c == 1)
    def _():
        pltpu.sync_copy(dst2.at[pl.ds(s * NCHT, NCHT)], idx_v)
    @pl.when(s < 10)
    def _():
        pltpu.sync_copy(z1.at[pl.ds(s * 1000, 1000)], zv)
        pltpu.sync_copy(zv, deg_s.at[pl.ds(s * 1000, 1000)])
    plsc.subcore_barrier()

    @pl.loop(0, NCHT, step=DEG_G)
    def _(jb):
        for g in range(DEG_G):
            pltpu.async_copy(ones_r, deg_s.at[idx_v.at[jb + g]], sem, add=True)
        for g in range(DEG_G):
            pltpu.make_async_copy(ones_r, deg_s.at[idx_v.at[jb + g]], sem).wait()

    plsc.subcore_barrier()

    @pl.when(s == 0)
    def _():
        pltpu.sync_copy(deg_s, dv)
        pltpu.sync_copy(dv, deg_out.at[pl.ds(c * N, N)])


ZT = 10
RPZ = N // ZT
SRW = 200
NBUF = 4
PF = 2
NQ = 1
QCH = NCHT // NQ


@functools.partial(
    pl.kernel,
    out_type=jax.ShapeDtypeStruct((NC * N, DH), jnp.float32),
    mesh=_sc_mesh,
    scratch_types=[
        pltpu.VMEM((QCH, CHA), jnp.int32),
        pltpu.VMEM((QCH, CHA), jnp.int32),
        pltpu.VMEM((NBUF, CHA, DH), jnp.float32),
        pltpu.VMEM((SRW, DH), jnp.float32),
        pltpu.VMEM_SHARED((N, DH), jnp.float32),
    ] + [pltpu.SemaphoreType.DMA] * (2 * NBUF),
    compiler_params=pltpu.CompilerParams(use_tc_tiling_on_sc=False),
)
def _agg_kernel(feat_lo, feat_hi, src2, dst2, zh, part,
                sidx_v, didx_v, rows_v, stage_v, agg_s, *sems):
    gsem = sems[:NBUF]
    ssem = sems[NBUF:]
    c = lax.axis_index("c")
    s = lax.axis_index("s")
    @pl.when(s < ZT)
    def _():
        for r in range(RPZ // SRW):
            off = pl.ds(s * RPZ + r * SRW, SRW)
            pltpu.sync_copy(zh.at[off], stage_v)
            pltpu.sync_copy(stage_v, agg_s.at[off])
    plsc.subcore_barrier()

    def edge_pass(ftab):
        def start_gather(j, b):
            pltpu.async_copy(ftab.at[sidx_v.at[j]], rows_v.at[b], gsem[b])

        def wait_gather(j, b):
            pltpu.make_async_copy(ftab.at[sidx_v.at[j]], rows_v.at[b],
                                  gsem[b]).wait()

        def start_scatter(j, b):
            pltpu.async_copy(rows_v.at[b], agg_s.at[didx_v.at[j]], ssem[b],
                             add=True)

        def wait_scatter(j, b):
            pltpu.make_async_copy(rows_v.at[b], agg_s.at[didx_v.at[j]],
                                  ssem[b]).wait()

        for q in range(NQ):
            qoff = pl.ds(s * NCHT + q * QCH, QCH)
            pltpu.sync_copy(src2.at[qoff], sidx_v)
            pltpu.sync_copy(dst2.at[qoff], didx_v)

            for b in range(PF):
                start_gather(b, b)

            @pl.loop(0, QCH, step=NBUF)
            def _(jb):
                for bb in range(NBUF):
                    j = jb + bb
                    nb = (bb + PF) % NBUF
                    @pl.when(j + PF < QCH)
                    def _(j=j, nb=nb):
                        @pl.when(j + PF - NBUF >= 0)
                        def _():
                            wait_scatter(j + PF - NBUF, nb)
                        start_gather(j + PF, nb)
                    wait_gather(j, bb)
                    start_scatter(j, bb)

            for bb in range(NBUF):
                j = QCH - NBUF + bb
                wait_scatter(j, j % NBUF)

    @pl.when(c == 0)
    def _():
        edge_pass(feat_lo)

    @pl.when(c == 1)
    def _():
        edge_pass(feat_hi)

    plsc.subcore_barrier()

    @pl.when(s < ZT)
    def _():
        for r in range(RPZ // SRW):
            pltpu.sync_copy(agg_s.at[pl.ds(s * RPZ + r * SRW, SRW)], stage_v)
            pltpu.sync_copy(stage_v,
                            part.at[pl.ds(c * N + s * RPZ + r * SRW, SRW)])


def _feat_body(x_ref, dsrc_ref, ddst_ref, flo_ref, fhi_ref, ndst_ref):
    nsrc = lax.rsqrt(jnp.maximum(dsrc_ref[...], 1.0))
    flo_ref[...] = x_ref[:, :DH] * nsrc
    fhi_ref[...] = x_ref[:, DH:] * nsrc
    ndst_ref[...] = lax.rsqrt(jnp.maximum(ddst_ref[...], 1.0))


_feat_call = pl.pallas_call(
    _feat_body,
    out_shape=[
        jax.ShapeDtypeStruct((N, DH), jnp.float32),
        jax.ShapeDtypeStruct((N, DH), jnp.float32),
        jax.ShapeDtypeStruct((N, 1), jnp.float32),
    ],
)


def _final_body(p_ref, w_ref, b_ref, x_ref, ndst_ref, g_ref, be_ref, o_ref):
    rst = (jnp.dot(p_ref[0], w_ref[0], preferred_element_type=jnp.float32)
           + jnp.dot(p_ref[1], w_ref[1], preferred_element_type=jnp.float32))
    rst = rst * ndst_ref[...] + b_ref[...]
    mean = jnp.mean(rst, axis=0, keepdims=True)
    var = jnp.mean(rst * rst, axis=0, keepdims=True) - mean * mean
    o_ref[...] = ((rst - mean) * lax.rsqrt(var + EPS) * g_ref[...]
                  + be_ref[...] + x_ref[...])


_final_call = pl.pallas_call(
    _final_body,
    out_shape=jax.ShapeDtypeStruct((N, D), jnp.float32),
)


def kernel(x, edge_index, W, b, gamma, beta):
    src2 = edge_index[0].reshape(E // CHA, CHA)
    dst2 = edge_index[1].reshape(E // CHA, CHA)
    z1 = jnp.zeros((N,), jnp.float32)
    zh = jnp.zeros((N, DH), jnp.float32)

    deg = _deg_kernel(src2, dst2, z1)
    dsrc = deg[:N].reshape(N, 1)
    ddst = deg[N:].reshape(N, 1)
    flo, fhi, ndst = _feat_call(x, dsrc, ddst)
    part = _agg_kernel(flo, fhi, src2, dst2, zh).reshape(NC, N, DH)
    w2 = W.reshape(2, DH, D)
    return _final_call(part, w2, b.reshape(1, D), x, ndst,
                       gamma.reshape(1, D), beta.reshape(1, D))
